# baseline (speedup 1.0000x reference)
import functools

import jax
import jax.numpy as jnp
from jax import lax
from jax.experimental import pallas as pl
from jax.experimental.pallas import tpu as pltpu
from jax.experimental.pallas import tpu_sc as plsc

_EMB = 64
_ROWS = 17
_PTAB = _ROWS * _ROWS
_NC, _NS = 2, 16
_NW = _NC * _NS
_SUB = 128
_GPC = 2
_CHUNK = _SUB * _GPC


def _combine_body(layer_ref, dir_ref, out_ref):
    out_ref[...] = layer_ref[...] + dir_ref[...]


def _pair_idx_body(rows_per_worker, even_ref, odd_ref, out_ref):
    row = lax.broadcasted_iota(jnp.int32, even_ref.shape, 0)
    out_ref[...] = (
        even_ref[...] * _ROWS + odd_ref[...] + (row // rows_per_worker) * _PTAB
    )


def _sc_gather(n_pairs):
    per_worker = n_pairs // _NW
    n_chunks = per_worker // _CHUNK
    n_iter = n_chunks // 2
    mesh = plsc.VectorSubcoreMesh(core_axis_name="c", subcore_axis_name="s")

    @functools.partial(
        pl.kernel,
        out_type=jax.ShapeDtypeStruct((n_pairs, 2 * _EMB), jnp.float32),
        mesh=mesh,
        scratch_types=[
            pltpu.VMEM((2, _GPC, _SUB), jnp.int32),
            pltpu.VMEM((2, _CHUNK, 2 * _EMB), jnp.float32),
            pltpu.SemaphoreType.DMA,
            pltpu.SemaphoreType.DMA,
        ],
    )
    def k(table_hbm, idx_hbm, out_hbm, idx_v, rows_v, sem_a, sem_b):
        wid = lax.axis_index("s") * _NC + lax.axis_index("c")
        row0 = wid * (per_worker // _SUB)

        def scoped():
            def fire(chunk, buf, sem):
                crow = row0 + chunk * _GPC
                pltpu.sync_copy(idx_hbm.at[pl.ds(crow, _GPC)], idx_v.at[buf])
                for j in range(_GPC):
                    pltpu.async_copy(
                        table_hbm.at[idx_v.at[buf].at[j]],
                        rows_v.at[buf].at[pl.ds(j * _SUB, _SUB)],
                        sem,
                    )

            def drain_and_out(chunk, buf, sem):
                for j in range(_GPC):
                    pltpu.make_async_copy(
                        table_hbm.at[idx_v.at[buf].at[j]],
                        rows_v.at[buf].at[pl.ds(j * _SUB, _SUB)],
                        sem,
                    ).wait()
                pltpu.sync_copy(
                    rows_v.at[buf],
                    out_hbm.at[pl.ds((row0 + chunk * _GPC) * _SUB, _CHUNK)],
                )

            fire(0, 0, sem_a)

            def body(i, carry):
                g = 2 * i
                fire(g + 1, 1, sem_b)
                drain_and_out(g, 0, sem_a)

                @pl.when(i < n_iter - 1)
                def _prefetch():
                    fire(g + 2, 0, sem_a)

                drain_and_out(g + 1, 1, sem_b)
                return carry

            lax.fori_loop(0, n_iter, body, 0)

        scoped()

    return k


def kernel(metal_layer, layer_table, direction_table):
    b, h = metal_layer.shape
    n = b * h
    n_pairs = n // 2

    layer_pad = jnp.pad(layer_table, ((0, 18 - _ROWS), (0, 0)))
    dir_tiled = jnp.tile(direction_table, (9, 1))
    combined = pl.pallas_call(
        _combine_body,
        out_shape=jax.ShapeDtypeStruct((18, _EMB), jnp.float32),
    )(layer_pad, dir_tiled)[:_ROWS]

    left = jnp.repeat(combined, _ROWS, axis=0)
    right = jnp.tile(combined, (_ROWS, 1))
    pair_table = jnp.tile(jnp.concatenate([left, right], axis=1), (_NW, 1))

    me = metal_layer.reshape(n_pairs, 2)
    even = me[:, 0].reshape(n_pairs // _SUB, _SUB)
    odd = me[:, 1].reshape(n_pairs // _SUB, _SUB)
    pair_idx = pl.pallas_call(
        functools.partial(_pair_idx_body, n_pairs // _SUB // _NW),
        out_shape=jax.ShapeDtypeStruct((n_pairs // _SUB, _SUB), jnp.int32),
    )(even, odd)

    out = _sc_gather(n_pairs)(pair_table, pair_idx)
    return out.reshape(b, h, _EMB)


# speedup vs baseline: 1.4007x; 1.4007x over previous
import functools

import jax
import jax.numpy as jnp
from jax import lax
from jax.experimental import pallas as pl
from jax.experimental.pallas import tpu as pltpu
from jax.experimental.pallas import tpu_sc as plsc

_EMB = 64
_ROWS = 17
_PTAB = _ROWS * _ROWS
_NC, _NS = 2, 16
_NW = _NC * _NS
_SUB = 128
_GPC = 2
_CHUNK = _SUB * _GPC
_L = 16


def _combine_body(layer_ref, dir_ref, out_ref):
    out_ref[...] = layer_ref[...] + dir_ref[...]


def _sc_gather(n_pairs):
    per_worker = n_pairs // _NW
    n_chunks = per_worker // _CHUNK
    n_iter = n_chunks // 2
    mesh = plsc.VectorSubcoreMesh(core_axis_name="c", subcore_axis_name="s")

    @functools.partial(
        pl.kernel,
        out_type=jax.ShapeDtypeStruct((n_pairs, 2 * _EMB), jnp.float32),
        mesh=mesh,
        scratch_types=[
            pltpu.VMEM((2 * _CHUNK,), jnp.int32),
            pltpu.VMEM((2 * _CHUNK,), jnp.int32),
            pltpu.VMEM((2, _GPC, _SUB), jnp.int32),
            pltpu.VMEM((2, _CHUNK, 2 * _EMB), jnp.float32),
            pltpu.SemaphoreType.DMA,
            pltpu.SemaphoreType.DMA,
        ],
    )
    def k(table_hbm, raw_hbm, out_hbm, raw_a, raw_b, idx_v, rows_v, sem_a, sem_b):
        wid = lax.axis_index("s") * _NC + lax.axis_index("c")
        row0 = wid * (per_worker // _SUB)
        tab0 = wid * _PTAB

        def scoped():
            def fire(chunk, buf, sem):
                crow = row0 + chunk * _GPC
                raw = raw_a if buf == 0 else raw_b
                pltpu.sync_copy(
                    raw_hbm.at[pl.ds(crow * 2 * _SUB, 2 * _CHUNK)],
                    raw,
                )
                lane = lax.iota(jnp.int32, _L)
                shift_idx = jnp.minimum(lane + 1, _L - 1)
                comp_idx = (lane % (_L // 2)) * 2
                lo_half = lane < (_L // 2)

                def take(v, idx):
                    return lax.gather(
                        v,
                        idx[:, None],
                        dimension_numbers=lax.GatherDimensionNumbers(
                            offset_dims=(),
                            collapsed_slice_dims=(0,),
                            start_index_map=(0,),
                        ),
                        slice_sizes=(1,),
                        mode=lax.GatherScatterMode.PROMISE_IN_BOUNDS,
                    )

                for j in range(_GPC):
                    for t in range(_SUB // _L):
                        base = 2 * (j * _SUB + t * _L)
                        v0 = raw[pl.ds(base, _L)]
                        v1 = raw[pl.ds(base + _L, _L)]
                        q0 = v0 * _ROWS + take(v0, shift_idx)
                        q1 = v1 * _ROWS + take(v1, shift_idx)
                        p = jnp.where(lo_half, take(q0, comp_idx), take(q1, comp_idx))
                        idx_v[buf, j, pl.ds(t * _L, _L)] = p + tab0
                for j in range(_GPC):
                    pltpu.async_copy(
                        table_hbm.at[idx_v.at[buf].at[j]],
                        rows_v.at[buf].at[pl.ds(j * _SUB, _SUB)],
                        sem,
                    )

            def drain_and_out(chunk, buf, sem):
                for j in range(_GPC):
                    pltpu.make_async_copy(
                        table_hbm.at[idx_v.at[buf].at[j]],
                        rows_v.at[buf].at[pl.ds(j * _SUB, _SUB)],
                        sem,
                    ).wait()
                pltpu.sync_copy(
                    rows_v.at[buf],
                    out_hbm.at[pl.ds((row0 + chunk * _GPC) * _SUB, _CHUNK)],
                )

            fire(0, 0, sem_a)

            def body(i, carry):
                g = 2 * i
                fire(g + 1, 1, sem_b)
                drain_and_out(g, 0, sem_a)

                @pl.when(i < n_iter - 1)
                def _prefetch():
                    fire(g + 2, 0, sem_a)

                drain_and_out(g + 1, 1, sem_b)
                return carry

            lax.fori_loop(0, n_iter, body, 0)

        scoped()

    return k


def kernel(metal_layer, layer_table, direction_table):
    b, h = metal_layer.shape
    n = b * h
    n_pairs = n // 2

    layer_pad = jnp.pad(layer_table, ((0, 18 - _ROWS), (0, 0)))
    dir_tiled = jnp.tile(direction_table, (9, 1))
    combined = pl.pallas_call(
        _combine_body,
        out_shape=jax.ShapeDtypeStruct((18, _EMB), jnp.float32),
    )(layer_pad, dir_tiled)[:_ROWS]

    left = jnp.repeat(combined, _ROWS, axis=0)
    right = jnp.tile(combined, (_ROWS, 1))
    pair_table = jnp.tile(jnp.concatenate([left, right], axis=1), (_NW, 1))

    out = _sc_gather(n_pairs)(pair_table, metal_layer.reshape(n))
    return out.reshape(b, h, _EMB)
